# Initial kernel scaffold; baseline (speedup 1.0000x reference)
import jax
import jax.numpy as jnp
from jax.experimental import pallas as pl


def kernel(component_mole_frac, prediction, component_batch_batch):
    raise NotImplementedError("write your pallas kernel here")


# speedup vs baseline: 40.2219x; 40.2219x over previous
import functools

import jax
import jax.numpy as jnp
from jax import lax
from jax.experimental import pallas as pl
from jax.experimental.pallas import tpu as pltpu
from jax.experimental.pallas import tpu_sc as plsc

N = 65536
M = 16384
R_GAS = 8.31446261815324
T_K = 298.15
RT = R_GAS * T_K

NS = 16
L = 16
ROW = 128
E = N // NS
ROWS = E // ROW
MC = M // NS

_mesh = plsc.VectorSubcoreMesh(core_axis_name="c", subcore_axis_name="s")


@functools.partial(
    pl.kernel,
    out_type=jax.ShapeDtypeStruct((L,), jnp.float32),
    mesh=_mesh,
    scratch_types=[
        pltpu.VMEM((ROWS, ROW), jnp.int32),
        pltpu.VMEM((ROWS, ROW), jnp.float32),
        pltpu.VMEM((ROWS, ROW), jnp.float32),
        pltpu.VMEM((MC,), jnp.float32),
        pltpu.VMEM((MC,), jnp.float32),
        pltpu.VMEM((ROW,), jnp.float32),
        pltpu.VMEM((L,), jnp.float32),
        pltpu.VMEM((NS, L), jnp.float32),
        pltpu.VMEM_SHARED((M,), jnp.float32),
        pltpu.VMEM_SHARED((M,), jnp.float32),
        pltpu.VMEM_SHARED((NS, L), jnp.float32),
    ],
)
def _gd_loss_kernel(pred_hbm, ids_hbm, out_hbm,
                    ids_v, r_v, m_v, s_v, c_v, ones_v, vec_v, part_v,
                    accS, accC, parts):
    cid = lax.axis_index("c")
    sid = lax.axis_index("s")
    row0 = sid * ROWS

    pltpu.sync_copy(ids_hbm.at[pl.ds(row0, ROWS)], ids_v)
    pltpu.sync_copy(pred_hbm.at[pl.ds(row0, ROWS)], r_v)

    for j in range(ROWS):
        for k in range(ROW // L):
            g = r_v[j, pl.ds(k * L, L)] * RT
            grad = g
            r_v[j, pl.ds(k * L, L)] = grad - g

    zero = jnp.zeros((L,), jnp.float32)
    for k in range(MC // L):
        s_v[pl.ds(k * L, L)] = zero
    pltpu.sync_copy(s_v, accS.at[pl.ds(sid * MC, MC)])
    pltpu.sync_copy(s_v, accC.at[pl.ds(sid * MC, MC)])
    one = jnp.full((L,), 1.0, jnp.float32)
    for k in range(ROW // L):
        ones_v[pl.ds(k * L, L)] = one
    plsc.subcore_barrier()

    for j in range(ROWS):
        pltpu.sync_copy(r_v.at[j], accS.at[ids_v.at[j]], add=True)
        pltpu.sync_copy(ones_v, accC.at[ids_v.at[j]], add=True)
    plsc.subcore_barrier()

    pltpu.sync_copy(accS.at[pl.ds(sid * MC, MC)], s_v)
    pltpu.sync_copy(accC.at[pl.ds(sid * MC, MC)], c_v)
    for k in range(MC // L):
        s = s_v[pl.ds(k * L, L)]
        c = c_v[pl.ds(k * L, L)]
        s_v[pl.ds(k * L, L)] = s / jnp.maximum(c, 1.0)
    pltpu.sync_copy(s_v, accS.at[pl.ds(sid * MC, MC)])
    plsc.subcore_barrier()

    for j in range(ROWS):
        pltpu.sync_copy(accS.at[ids_v.at[j]], m_v.at[j])
    acc = jnp.zeros((L,), jnp.float32)
    for j in range(ROWS):
        for k in range(ROW // L):
            d = r_v[j, pl.ds(k * L, L)] - m_v[j, pl.ds(k * L, L)]
            acc = acc + d * d
    vec_v[...] = acc
    pltpu.sync_copy(vec_v, parts.at[sid])
    plsc.subcore_barrier()

    @pl.when(jnp.logical_and(cid == 0, sid == 0))
    def _():
        pltpu.sync_copy(parts, part_v)
        tot = jnp.zeros((L,), jnp.float32)
        for j in range(NS):
            tot = tot + part_v[j, pl.ds(0, L)]
        loss = jnp.float32(0.0)
        for i in range(L):
            loss = loss + tot[i]
        loss = loss * (1.0 / M)
        vec_v[...] = jnp.full((L,), loss, jnp.float32)
        pltpu.sync_copy(vec_v, out_hbm)


def kernel(component_mole_frac, prediction, component_batch_batch):
    del component_mole_frac
    pred = prediction.reshape(N // ROW, ROW)
    ids = component_batch_batch.astype(jnp.int32).reshape(N // ROW, ROW)
    out = _gd_loss_kernel(pred, ids)
    return out[0]


# speedup vs baseline: 50.1182x; 1.2460x over previous
import functools

import jax
import jax.numpy as jnp
from jax import lax
from jax.experimental import pallas as pl
from jax.experimental.pallas import tpu as pltpu
from jax.experimental.pallas import tpu_sc as plsc

N = 65536
M = 16384
R_GAS = 8.31446261815324
T_K = 298.15
RT = R_GAS * T_K

NS = 16
L = 16
ROW = 128
E = N // NS
ROWS = E // ROW
MC = M // NS

_mesh = plsc.VectorSubcoreMesh(
    core_axis_name="c", subcore_axis_name="s", num_cores=1
)


@functools.partial(
    pl.kernel,
    out_type=jax.ShapeDtypeStruct((L,), jnp.float32),
    mesh=_mesh,
    scratch_types=[
        pltpu.VMEM((ROWS, ROW), jnp.int32),
        pltpu.VMEM((ROWS, ROW), jnp.float32),
        pltpu.VMEM((ROWS, ROW), jnp.float32),
        pltpu.VMEM((MC,), jnp.float32),
        pltpu.VMEM((MC,), jnp.float32),
        pltpu.VMEM((ROW,), jnp.float32),
        pltpu.VMEM((L,), jnp.float32),
        pltpu.VMEM((NS, L), jnp.float32),
        pltpu.VMEM_SHARED((M,), jnp.float32),
        pltpu.VMEM_SHARED((M,), jnp.float32),
        pltpu.VMEM_SHARED((NS, L), jnp.float32),
        pltpu.SemaphoreType.DMA,
    ],
)
def _gd_loss_kernel(pred_hbm, ids_hbm, out_hbm,
                    ids_v, r_v, m_v, s_v, c_v, ones_v, vec_v, part_v,
                    accS, accC, parts, sem):
    cid = lax.axis_index("c")
    sid = lax.axis_index("s")
    row0 = sid * ROWS

    pltpu.sync_copy(ids_hbm.at[pl.ds(row0, ROWS)], ids_v)
    pltpu.sync_copy(pred_hbm.at[pl.ds(row0, ROWS)], r_v)

    for j in range(ROWS):
        for k in range(ROW // L):
            g = r_v[j, pl.ds(k * L, L)] * RT
            grad = g
            r_v[j, pl.ds(k * L, L)] = grad - g

    zero = jnp.zeros((L,), jnp.float32)
    for k in range(MC // L):
        s_v[pl.ds(k * L, L)] = zero
    pltpu.sync_copy(s_v, accS.at[pl.ds(sid * MC, MC)])
    pltpu.sync_copy(s_v, accC.at[pl.ds(sid * MC, MC)])
    one = jnp.full((L,), 1.0, jnp.float32)
    for k in range(ROW // L):
        ones_v[pl.ds(k * L, L)] = one
    plsc.subcore_barrier()

    descs = []
    for j in range(ROWS):
        descs.append(pltpu.async_copy(r_v.at[j], accS.at[ids_v.at[j]], sem,
                                      add=True))
        descs.append(pltpu.async_copy(ones_v, accC.at[ids_v.at[j]], sem,
                                      add=True))
    for d in descs:
        d.wait()
    plsc.subcore_barrier()

    pltpu.sync_copy(accS.at[pl.ds(sid * MC, MC)], s_v)
    pltpu.sync_copy(accC.at[pl.ds(sid * MC, MC)], c_v)
    for k in range(MC // L):
        s = s_v[pl.ds(k * L, L)]
        c = c_v[pl.ds(k * L, L)]
        s_v[pl.ds(k * L, L)] = s / jnp.maximum(c, 1.0)
    pltpu.sync_copy(s_v, accS.at[pl.ds(sid * MC, MC)])
    plsc.subcore_barrier()

    descs = []
    for j in range(ROWS):
        descs.append(pltpu.async_copy(accS.at[ids_v.at[j]], m_v.at[j], sem))
    for d in descs:
        d.wait()
    acc = jnp.zeros((L,), jnp.float32)
    for j in range(ROWS):
        for k in range(ROW // L):
            d = r_v[j, pl.ds(k * L, L)] - m_v[j, pl.ds(k * L, L)]
            acc = acc + d * d
    vec_v[...] = acc
    pltpu.sync_copy(vec_v, parts.at[sid])
    plsc.subcore_barrier()

    @pl.when(jnp.logical_and(cid == 0, sid == 0))
    def _():
        pltpu.sync_copy(parts, part_v)
        tot = jnp.zeros((L,), jnp.float32)
        for j in range(NS):
            tot = tot + part_v[j, pl.ds(0, L)]
        loss = jnp.float32(0.0)
        for i in range(L):
            loss = loss + tot[i]
        loss = loss * (1.0 / M)
        vec_v[...] = jnp.full((L,), loss, jnp.float32)
        pltpu.sync_copy(vec_v, out_hbm)


def kernel(component_mole_frac, prediction, component_batch_batch):
    del component_mole_frac
    pred = prediction.reshape(N // ROW, ROW)
    ids = component_batch_batch.astype(jnp.int32).reshape(N // ROW, ROW)
    out = _gd_loss_kernel(pred, ids)
    return out[0]


# speedup vs baseline: 55.7927x; 1.1132x over previous
import functools

import jax
import jax.numpy as jnp
from jax import lax
from jax.experimental import pallas as pl
from jax.experimental.pallas import tpu as pltpu
from jax.experimental.pallas import tpu_sc as plsc

N = 65536
M = 16384
R_GAS = 8.31446261815324
T_K = 298.15
RT = R_GAS * T_K

NS = 16
L = 16
ROW = 128
E = N // NS
ROWS = E // ROW
MC = M // NS

_mesh = plsc.VectorSubcoreMesh(
    core_axis_name="c", subcore_axis_name="s", num_cores=1
)


@functools.partial(
    pl.kernel,
    out_type=jax.ShapeDtypeStruct((L,), jnp.float32),
    mesh=_mesh,
    scratch_types=[
        pltpu.VMEM((ROWS, ROW), jnp.int32),
        pltpu.VMEM((ROWS, ROW), jnp.float32),
        pltpu.VMEM((MC,), jnp.float32),
        pltpu.VMEM((MC,), jnp.float32),
        pltpu.VMEM((MC,), jnp.float32),
        pltpu.VMEM((ROW,), jnp.float32),
        pltpu.VMEM((L,), jnp.float32),
        pltpu.VMEM((NS, L), jnp.float32),
        pltpu.VMEM_SHARED((M,), jnp.float32),
        pltpu.VMEM_SHARED((M,), jnp.float32),
        pltpu.VMEM_SHARED((NS, L), jnp.float32),
        pltpu.SemaphoreType.DMA,
    ],
)
def _gd_loss_kernel(pred_hbm, ids_hbm, out_hbm,
                    ids_v, r_v, s_v, c_v, z_v, ones_v, vec_v, part_v,
                    accS, accC, parts, sem):
    sid = lax.axis_index("s")
    row0 = sid * ROWS

    pltpu.sync_copy(ids_hbm.at[pl.ds(row0, ROWS)], ids_v)
    pltpu.sync_copy(pred_hbm.at[pl.ds(row0, ROWS)], r_v)
    zero = jnp.zeros((L,), jnp.float32)
    for k in range(MC // L):
        z_v[pl.ds(k * L, L)] = zero
    one = jnp.full((L,), 1.0, jnp.float32)
    for k in range(ROW // L):
        ones_v[pl.ds(k * L, L)] = one
    pltpu.sync_copy(z_v, accS.at[pl.ds(sid * MC, MC)])
    pltpu.sync_copy(z_v, accC.at[pl.ds(sid * MC, MC)])

    acc = jnp.zeros((L,), jnp.float32)
    for j in range(ROWS):
        for k in range(ROW // L):
            g = r_v[j, pl.ds(k * L, L)] * RT
            grad = g
            r = grad - g
            r_v[j, pl.ds(k * L, L)] = r
            acc = acc + r * r
    plsc.subcore_barrier()

    descs = []
    for j in range(ROWS):
        descs.append(pltpu.async_copy(r_v.at[j], accS.at[ids_v.at[j]], sem,
                                      add=True))
        descs.append(pltpu.async_copy(ones_v, accC.at[ids_v.at[j]], sem,
                                      add=True))
    for d in descs:
        d.wait()
    plsc.subcore_barrier()

    pltpu.sync_copy(accS.at[pl.ds(sid * MC, MC)], s_v)
    pltpu.sync_copy(accC.at[pl.ds(sid * MC, MC)], c_v)
    for k in range(MC // L):
        s = s_v[pl.ds(k * L, L)]
        c = c_v[pl.ds(k * L, L)]
        acc = acc - s * s / jnp.maximum(c, 1.0)
    vec_v[...] = acc
    pltpu.sync_copy(vec_v, parts.at[sid])
    plsc.subcore_barrier()

    @pl.when(sid == 0)
    def _():
        pltpu.sync_copy(parts, part_v)
        tot = jnp.zeros((L,), jnp.float32)
        for j in range(NS):
            tot = tot + part_v[j, pl.ds(0, L)]
        loss = jnp.float32(0.0)
        for i in range(L):
            loss = loss + tot[i]
        loss = loss * (1.0 / M)
        vec_v[...] = jnp.full((L,), loss, jnp.float32)
        pltpu.sync_copy(vec_v, out_hbm)


def kernel(component_mole_frac, prediction, component_batch_batch):
    del component_mole_frac
    pred = prediction.reshape(N // ROW, ROW)
    ids = component_batch_batch.astype(jnp.int32).reshape(N // ROW, ROW)
    out = _gd_loss_kernel(pred, ids)
    return out[0]


# speedup vs baseline: 56.0710x; 1.0050x over previous
import functools

import jax
import jax.numpy as jnp
from jax import lax
from jax.experimental import pallas as pl
from jax.experimental.pallas import tpu as pltpu
from jax.experimental.pallas import tpu_sc as plsc

N = 65536
M = 16384
R_GAS = 8.31446261815324
T_K = 298.15
RT = R_GAS * T_K

NS = 16
L = 16
ROW = 128
E = N // NS
ROWS = E // ROW
MC = M // NS

_mesh = plsc.VectorSubcoreMesh(
    core_axis_name="c", subcore_axis_name="s", num_cores=1
)


@functools.partial(
    pl.kernel,
    out_type=jax.ShapeDtypeStruct((L,), jnp.float32),
    mesh=_mesh,
    scratch_types=[
        pltpu.VMEM((ROWS, ROW), jnp.int32),
        pltpu.VMEM((ROWS, ROW), jnp.float32),
        pltpu.VMEM((MC,), jnp.float32),
        pltpu.VMEM((MC,), jnp.float32),
        pltpu.VMEM((MC,), jnp.float32),
        pltpu.VMEM((ROW,), jnp.float32),
        pltpu.VMEM((L,), jnp.float32),
        pltpu.VMEM((NS, L), jnp.float32),
        pltpu.VMEM_SHARED((M,), jnp.float32),
        pltpu.VMEM_SHARED((M,), jnp.float32),
        pltpu.VMEM_SHARED((NS, L), jnp.float32),
        pltpu.SemaphoreType.DMA,
    ],
)
def _gd_loss_kernel(pred_hbm, ids_hbm, out_hbm,
                    ids_v, r_v, s_v, c_v, z_v, ones_v, vec_v, part_v,
                    accS, accC, parts, sem):
    sid = lax.axis_index("s")
    row0 = sid * ROWS

    pltpu.sync_copy(ids_hbm.at[pl.ds(row0, ROWS)], ids_v)
    pltpu.sync_copy(pred_hbm.at[pl.ds(row0, ROWS)], r_v)
    zero = jnp.zeros((L,), jnp.float32)
    for k in range(MC // L):
        z_v[pl.ds(k * L, L)] = zero
    one = jnp.full((L,), 1.0, jnp.float32)
    for k in range(ROW // L):
        ones_v[pl.ds(k * L, L)] = one
    pltpu.sync_copy(z_v, accS.at[pl.ds(sid * MC, MC)])
    pltpu.sync_copy(z_v, accC.at[pl.ds(sid * MC, MC)])

    acc = jnp.zeros((L,), jnp.float32)
    plsc.subcore_barrier()

    descs = []
    for j in range(ROWS):
        for k in range(ROW // L):
            g = r_v[j, pl.ds(k * L, L)] * RT
            grad = g
            r = grad - g
            r_v[j, pl.ds(k * L, L)] = r
            acc = acc + r * r
        descs.append(pltpu.async_copy(r_v.at[j], accS.at[ids_v.at[j]], sem,
                                      add=True))
        descs.append(pltpu.async_copy(ones_v, accC.at[ids_v.at[j]], sem,
                                      add=True))
    for d in descs:
        d.wait()
    plsc.subcore_barrier()

    pltpu.sync_copy(accS.at[pl.ds(sid * MC, MC)], s_v)
    pltpu.sync_copy(accC.at[pl.ds(sid * MC, MC)], c_v)
    for k in range(MC // L):
        s = s_v[pl.ds(k * L, L)]
        c = c_v[pl.ds(k * L, L)]
        acc = acc - s * s / jnp.maximum(c, 1.0)
    vec_v[...] = acc
    pltpu.sync_copy(vec_v, parts.at[sid])
    plsc.subcore_barrier()

    @pl.when(sid == 0)
    def _():
        pltpu.sync_copy(parts, part_v)
        tot = jnp.zeros((L,), jnp.float32)
        for j in range(NS):
            tot = tot + part_v[j, pl.ds(0, L)]
        loss = jnp.float32(0.0)
        for i in range(L):
            loss = loss + tot[i]
        loss = loss * (1.0 / M)
        vec_v[...] = jnp.full((L,), loss, jnp.float32)
        pltpu.sync_copy(vec_v, out_hbm)


def kernel(component_mole_frac, prediction, component_batch_batch):
    del component_mole_frac
    pred = prediction.reshape(N // ROW, ROW)
    ids = component_batch_batch.astype(jnp.int32).reshape(N // ROW, ROW)
    out = _gd_loss_kernel(pred, ids)
    return out[0]


# speedup vs baseline: 57.8512x; 1.0317x over previous
import functools

import jax
import jax.numpy as jnp
from jax import lax
from jax.experimental import pallas as pl
from jax.experimental.pallas import tpu as pltpu
from jax.experimental.pallas import tpu_sc as plsc

N = 65536
M = 16384
R_GAS = 8.31446261815324
T_K = 298.15
RT = R_GAS * T_K

NS = 16
L = 16
ROW = 128
E = N // NS
ROWS = E // ROW
MC = M // NS

_mesh = plsc.VectorSubcoreMesh(
    core_axis_name="c", subcore_axis_name="s", num_cores=1
)


@functools.partial(
    pl.kernel,
    out_type=jax.ShapeDtypeStruct((L,), jnp.float32),
    mesh=_mesh,
    scratch_types=[
        pltpu.VMEM((ROWS, ROW), jnp.int32),
        pltpu.VMEM((ROWS, ROW), jnp.float32),
        pltpu.VMEM((MC,), jnp.float32),
        pltpu.VMEM((MC,), jnp.float32),
        pltpu.VMEM((MC,), jnp.float32),
        pltpu.VMEM((ROW,), jnp.float32),
        pltpu.VMEM((L,), jnp.float32),
        pltpu.VMEM((NS, L), jnp.float32),
        pltpu.VMEM_SHARED((M,), jnp.float32),
        pltpu.VMEM_SHARED((M,), jnp.float32),
        pltpu.VMEM_SHARED((NS, L), jnp.float32),
        pltpu.SemaphoreType.DMA,
    ],
)
def _gd_loss_kernel(pred_hbm, ids_hbm, out_hbm,
                    ids_v, r_v, s_v, c_v, z_v, ones_v, vec_v, part_v,
                    accS, accC, parts, sem):
    sid = lax.axis_index("s")
    row0 = sid * ROWS

    loads = [
        pltpu.async_copy(ids_hbm.at[pl.ds(row0, ROWS)], ids_v, sem),
        pltpu.async_copy(pred_hbm.at[pl.ds(row0, ROWS)], r_v, sem),
    ]
    zero = jnp.zeros((L,), jnp.float32)
    for k in range(MC // L):
        z_v[pl.ds(k * L, L)] = zero
    one = jnp.full((L,), 1.0, jnp.float32)
    for k in range(ROW // L):
        ones_v[pl.ds(k * L, L)] = one
    pltpu.sync_copy(z_v, accS.at[pl.ds(sid * MC, MC)])
    pltpu.sync_copy(z_v, accC.at[pl.ds(sid * MC, MC)])
    for d in loads:
        d.wait()

    acc = jnp.zeros((L,), jnp.float32)
    plsc.subcore_barrier()

    descs = []
    for j in range(ROWS):
        for k in range(ROW // L):
            g = r_v[j, pl.ds(k * L, L)] * RT
            grad = g
            r = grad - g
            r_v[j, pl.ds(k * L, L)] = r
            acc = acc + r * r
        descs.append(pltpu.async_copy(r_v.at[j], accS.at[ids_v.at[j]], sem,
                                      add=True))
        descs.append(pltpu.async_copy(ones_v, accC.at[ids_v.at[j]], sem,
                                      add=True))
    for d in descs:
        d.wait()
    plsc.subcore_barrier()

    pltpu.sync_copy(accS.at[pl.ds(sid * MC, MC)], s_v)
    pltpu.sync_copy(accC.at[pl.ds(sid * MC, MC)], c_v)
    for k in range(MC // L):
        s = s_v[pl.ds(k * L, L)]
        c = c_v[pl.ds(k * L, L)]
        acc = acc - s * s / jnp.maximum(c, 1.0)
    vec_v[...] = acc
    pltpu.sync_copy(vec_v, parts.at[sid])
    plsc.subcore_barrier()

    @pl.when(sid == 0)
    def _():
        pltpu.sync_copy(parts, part_v)
        tot = jnp.zeros((L,), jnp.float32)
        for j in range(NS):
            tot = tot + part_v[j, pl.ds(0, L)]
        loss = jnp.float32(0.0)
        for i in range(L):
            loss = loss + tot[i]
        loss = loss * (1.0 / M)
        vec_v[...] = jnp.full((L,), loss, jnp.float32)
        pltpu.sync_copy(vec_v, out_hbm)


def kernel(component_mole_frac, prediction, component_batch_batch):
    del component_mole_frac
    pred = prediction.reshape(N // ROW, ROW)
    ids = component_batch_batch.astype(jnp.int32).reshape(N // ROW, ROW)
    out = _gd_loss_kernel(pred, ids)
    return out[0]
